# baseline (speedup 1.0000x reference)
import jax
import jax.numpy as jnp
from jax import lax
from jax.experimental import pallas as pl

_N_BLOCK = 1024
_MAX = 128


def _body(feats_ref, w1_ref, b1_ref, w2_ref, b2_ref,
          pts_ref, z_ref, out_ref):
    f = feats_ref[...]
    near = f[:, 6:7]
    far = f[:, 7:8]
    dist = f[:, 8:9]

    ns = jnp.where(dist < 25.0, 128,
         jnp.where(dist < 50.0, 64,
         jnp.where(dist < 100.0, 32, 16))).astype(jnp.int32)
    inv = jnp.where(dist < 25.0, 1.0 / 127.0,
          jnp.where(dist < 50.0, 1.0 / 63.0,
          jnp.where(dist < 100.0, 1.0 / 31.0, 1.0 / 15.0)))

    R = f.shape[0]

    j = lax.broadcasted_iota(jnp.int32, (R, _MAX), 1)
    t = j.astype(jnp.float32) * inv
    z = near * (1.0 - t) + far * t
    z_ref[...] = jnp.where(j < ns, z, 0.0)

    k = lax.broadcasted_iota(jnp.int32, (R, 3 * _MAX), 1)
    j3 = k // 3
    c = k - 3 * j3
    t3 = j3.astype(jnp.float32) * inv
    z3 = near * (1.0 - t3) + far * t3
    o_sel = jnp.where(c == 0, f[:, 0:1], jnp.where(c == 1, f[:, 1:2], f[:, 2:3]))
    d_sel = jnp.where(c == 0, f[:, 3:4], jnp.where(c == 1, f[:, 4:5], f[:, 5:6]))
    pts_ref[...] = jnp.where(j3 < ns, o_sel + d_sel * z3, 0.0)

    h = jnp.maximum(
        jnp.dot(f, w1_ref[...], preferred_element_type=jnp.float32) + b1_ref[...],
        0.0)
    out_ref[...] = jnp.dot(h, w2_ref[...], preferred_element_type=jnp.float32) + b2_ref[...]


def kernel(rays_o, rays_d, bounds, distances, W1, b1, W2, b2):
    N = rays_o.shape[0]
    feats = jnp.concatenate([rays_o, rays_d, bounds, distances[:, None]], axis=-1)
    R = _N_BLOCK
    grid = (N // R,)

    pts_flat, z_vals, model_out = pl.pallas_call(
        _body,
        grid=grid,
        in_specs=[
            pl.BlockSpec((R, 9), lambda i: (i, 0)),
            pl.BlockSpec((9, 256), lambda i: (0, 0)),
            pl.BlockSpec((1, 256), lambda i: (0, 0)),
            pl.BlockSpec((256, 4), lambda i: (0, 0)),
            pl.BlockSpec((1, 4), lambda i: (0, 0)),
        ],
        out_specs=[
            pl.BlockSpec((R, 3 * _MAX), lambda i: (i, 0)),
            pl.BlockSpec((R, _MAX), lambda i: (i, 0)),
            pl.BlockSpec((R, 4), lambda i: (i, 0)),
        ],
        out_shape=[
            jax.ShapeDtypeStruct((N, 3 * _MAX), jnp.float32),
            jax.ShapeDtypeStruct((N, _MAX), jnp.float32),
            jax.ShapeDtypeStruct((N, 4), jnp.float32),
        ],
    )(feats, W1, b1.reshape(1, 256), W2, b2.reshape(1, 4))

    return pts_flat.reshape(N, _MAX, 3), z_vals, model_out


# speedup vs baseline: 3.4948x; 3.4948x over previous
import jax
import jax.numpy as jnp
from jax import lax
from jax.experimental import pallas as pl

_N_BLOCK = 1024
_MAX = 128


def _body(feats_ref, featst_ref, w1t_ref, b1_ref, w2t_ref, b2_ref,
          pts_ref, z_ref, outt_ref):
    f = feats_ref[...]
    near = f[:, 6:7]
    far = f[:, 7:8]
    dist = f[:, 8:9]

    ns = jnp.where(dist < 25.0, 128,
         jnp.where(dist < 50.0, 64,
         jnp.where(dist < 100.0, 32, 16))).astype(jnp.int32)
    inv = jnp.where(dist < 25.0, 1.0 / 127.0,
          jnp.where(dist < 50.0, 1.0 / 63.0,
          jnp.where(dist < 100.0, 1.0 / 31.0, 1.0 / 15.0)))

    R = f.shape[0]

    j = lax.broadcasted_iota(jnp.int32, (R, _MAX), 1)
    live = j < ns
    t = j.astype(jnp.float32) * inv
    z = near * (1.0 - t) + far * t
    z_ref[...] = jnp.where(live, z, 0.0)

    for c in range(3):
        pts_ref[c, :, :] = jnp.where(
            live, f[:, c:c + 1] + f[:, c + 3:c + 4] * z, 0.0)

    ft = featst_ref[...]
    h = jnp.maximum(
        jnp.dot(w1t_ref[...], ft, preferred_element_type=jnp.float32) + b1_ref[...],
        0.0)
    outt_ref[...] = jnp.dot(w2t_ref[...], h, preferred_element_type=jnp.float32) + b2_ref[...]


def kernel(rays_o, rays_d, bounds, distances, W1, b1, W2, b2):
    N = rays_o.shape[0]
    feats = jnp.concatenate([rays_o, rays_d, bounds, distances[:, None]], axis=-1)
    feats_t = jnp.concatenate([rays_o.T, rays_d.T, bounds.T, distances[None, :]], axis=0)
    R = _N_BLOCK
    grid = (N // R,)

    pts_t, z_vals, out_t = pl.pallas_call(
        _body,
        grid=grid,
        in_specs=[
            pl.BlockSpec((R, 9), lambda i: (i, 0)),
            pl.BlockSpec((9, R), lambda i: (0, i)),
            pl.BlockSpec((256, 9), lambda i: (0, 0)),
            pl.BlockSpec((256, 1), lambda i: (0, 0)),
            pl.BlockSpec((4, 256), lambda i: (0, 0)),
            pl.BlockSpec((4, 1), lambda i: (0, 0)),
        ],
        out_specs=[
            pl.BlockSpec((3, R, _MAX), lambda i: (0, i, 0)),
            pl.BlockSpec((R, _MAX), lambda i: (i, 0)),
            pl.BlockSpec((4, R), lambda i: (0, i)),
        ],
        out_shape=[
            jax.ShapeDtypeStruct((3, N, _MAX), jnp.float32),
            jax.ShapeDtypeStruct((N, _MAX), jnp.float32),
            jax.ShapeDtypeStruct((4, N), jnp.float32),
        ],
    )(feats, feats_t, W1.T, b1.reshape(256, 1), W2.T, b2.reshape(4, 1))

    return jnp.transpose(pts_t, (1, 2, 0)), z_vals, out_t.T


# speedup vs baseline: 4.9028x; 1.4029x over previous
import numpy as np
import jax
import jax.numpy as jnp
from jax import lax
from jax.experimental import pallas as pl

_N_BLOCK = 1024
_MAX = 128

_SEL = np.zeros((9, 9 * _MAX), dtype=np.float32)
for _g in range(9):
    _SEL[_g, _g * _MAX:(_g + 1) * _MAX] = 1.0
_SEL[6, 7 * _MAX:8 * _MAX] = -1.0


def _body(featst_ref, sel_ref, w1t_ref, b1_ref, w2t_ref, b2_ref,
          pts_ref, z_ref, outt_ref):
    ft = featst_ref[...]
    sel = sel_ref[...]
    b = lax.dot_general(ft, sel, (((0,), (0,)), ((), ())),
                        preferred_element_type=jnp.float32)
    near = b[:, 6 * _MAX:7 * _MAX]
    fmn = b[:, 7 * _MAX:8 * _MAX]
    dist = b[:, 8 * _MAX:9 * _MAX]

    R = ft.shape[1]
    jf = lax.broadcasted_iota(jnp.int32, (R, _MAX), 1).astype(jnp.float32)

    m25 = dist < 25.0
    m50 = dist < 50.0
    m100 = dist < 100.0
    inv = jnp.where(m25, 1.0 / 127.0,
          jnp.where(m50, 1.0 / 63.0,
          jnp.where(m100, 1.0 / 31.0, 1.0 / 15.0)))
    nsf = jnp.where(m25, 128.0,
          jnp.where(m50, 64.0,
          jnp.where(m100, 32.0, 16.0)))
    live = jf < nsf

    t = jf * inv
    z = near + fmn * t
    z_ref[...] = jnp.where(live, z, 0.0)

    for c in range(3):
        o_c = b[:, c * _MAX:(c + 1) * _MAX]
        d_c = b[:, (c + 3) * _MAX:(c + 4) * _MAX]
        pts_ref[c, :, :] = jnp.where(live, o_c + d_c * z, 0.0)

    h = jnp.maximum(
        jnp.dot(w1t_ref[...], ft, preferred_element_type=jnp.float32) + b1_ref[...],
        0.0)
    outt_ref[...] = jnp.dot(w2t_ref[...], h, preferred_element_type=jnp.float32) + b2_ref[...]


def kernel(rays_o, rays_d, bounds, distances, W1, b1, W2, b2):
    N = rays_o.shape[0]
    feats_t = jnp.concatenate([rays_o.T, rays_d.T, bounds.T, distances[None, :]], axis=0)
    R = _N_BLOCK
    grid = (N // R,)

    pts_t, z_vals, out_t = pl.pallas_call(
        _body,
        grid=grid,
        in_specs=[
            pl.BlockSpec((9, R), lambda i: (0, i)),
            pl.BlockSpec((9, 9 * _MAX), lambda i: (0, 0)),
            pl.BlockSpec((256, 9), lambda i: (0, 0)),
            pl.BlockSpec((256, 1), lambda i: (0, 0)),
            pl.BlockSpec((4, 256), lambda i: (0, 0)),
            pl.BlockSpec((4, 1), lambda i: (0, 0)),
        ],
        out_specs=[
            pl.BlockSpec((3, R, _MAX), lambda i: (0, i, 0)),
            pl.BlockSpec((R, _MAX), lambda i: (i, 0)),
            pl.BlockSpec((4, R), lambda i: (0, i)),
        ],
        out_shape=[
            jax.ShapeDtypeStruct((3, N, _MAX), jnp.float32),
            jax.ShapeDtypeStruct((N, _MAX), jnp.float32),
            jax.ShapeDtypeStruct((4, N), jnp.float32),
        ],
    )(feats_t, jnp.asarray(_SEL), W1.T, b1.reshape(256, 1), W2.T, b2.reshape(4, 1))

    return jnp.transpose(pts_t, (1, 2, 0)), z_vals, out_t.T


# speedup vs baseline: 5.5690x; 1.1359x over previous
import numpy as np
import jax
import jax.numpy as jnp
from jax import lax
from jax.experimental import pallas as pl

_N_BLOCK = 2048
_MAX = 128

_SEL = np.zeros((9, 9 * _MAX), dtype=np.float32)
for _g in range(9):
    _SEL[_g, _g * _MAX:(_g + 1) * _MAX] = 1.0
_SEL[6, 7 * _MAX:8 * _MAX] = -1.0


def _body(featst_ref, sel_ref, w1t_ref, b1_ref, w2t_ref, b2_ref,
          pts_ref, z_ref, outt_ref):
    ft = featst_ref[...]
    sel = sel_ref[...]
    b = lax.dot_general(ft, sel, (((0,), (0,)), ((), ())),
                        preferred_element_type=jnp.float32)
    near = b[:, 6 * _MAX:7 * _MAX]
    fmn = b[:, 7 * _MAX:8 * _MAX]
    dist = b[:, 8 * _MAX:9 * _MAX]

    R = ft.shape[1]
    jf = lax.broadcasted_iota(jnp.int32, (R, _MAX), 1).astype(jnp.float32)

    m25 = dist < 25.0
    m50 = dist < 50.0
    m100 = dist < 100.0
    inv = jnp.where(m25, 1.0 / 127.0,
          jnp.where(m50, 1.0 / 63.0,
          jnp.where(m100, 1.0 / 31.0, 1.0 / 15.0)))
    nsf = jnp.where(m25, 128.0,
          jnp.where(m50, 64.0,
          jnp.where(m100, 32.0, 16.0)))
    live = jf < nsf

    t = jf * inv
    z = near + fmn * t
    z_ref[...] = jnp.where(live, z, 0.0)

    for c in range(3):
        o_c = b[:, c * _MAX:(c + 1) * _MAX]
        d_c = b[:, (c + 3) * _MAX:(c + 4) * _MAX]
        pts_ref[c, :, :] = jnp.where(live, o_c + d_c * z, 0.0)

    h = jnp.maximum(
        jnp.dot(w1t_ref[...], ft, preferred_element_type=jnp.float32) + b1_ref[...],
        0.0)
    outt_ref[...] = jnp.dot(w2t_ref[...], h, preferred_element_type=jnp.float32) + b2_ref[...]


def kernel(rays_o, rays_d, bounds, distances, W1, b1, W2, b2):
    N = rays_o.shape[0]
    feats_t = jnp.concatenate([rays_o.T, rays_d.T, bounds.T, distances[None, :]], axis=0)
    R = _N_BLOCK
    grid = (N // R,)

    pts_t, z_vals, out_t = pl.pallas_call(
        _body,
        grid=grid,
        in_specs=[
            pl.BlockSpec((9, R), lambda i: (0, i)),
            pl.BlockSpec((9, 9 * _MAX), lambda i: (0, 0)),
            pl.BlockSpec((256, 9), lambda i: (0, 0)),
            pl.BlockSpec((256, 1), lambda i: (0, 0)),
            pl.BlockSpec((4, 256), lambda i: (0, 0)),
            pl.BlockSpec((4, 1), lambda i: (0, 0)),
        ],
        out_specs=[
            pl.BlockSpec((3, R, _MAX), lambda i: (0, i, 0)),
            pl.BlockSpec((R, _MAX), lambda i: (i, 0)),
            pl.BlockSpec((4, R), lambda i: (0, i)),
        ],
        out_shape=[
            jax.ShapeDtypeStruct((3, N, _MAX), jnp.float32),
            jax.ShapeDtypeStruct((N, _MAX), jnp.float32),
            jax.ShapeDtypeStruct((4, N), jnp.float32),
        ],
    )(feats_t, jnp.asarray(_SEL), W1.T, b1.reshape(256, 1), W2.T, b2.reshape(4, 1))

    return jnp.transpose(pts_t, (1, 2, 0)), z_vals, out_t.T


# speedup vs baseline: 5.7870x; 1.0391x over previous
import numpy as np
import jax
import jax.numpy as jnp
from jax import lax
from jax.experimental import pallas as pl

_N_BLOCK = 4096
_MAX = 128

_SEL = np.zeros((9, 9 * _MAX), dtype=np.float32)
for _g in range(9):
    _SEL[_g, _g * _MAX:(_g + 1) * _MAX] = 1.0
_SEL[6, 7 * _MAX:8 * _MAX] = -1.0


def _body(featst_ref, sel_ref, w1t_ref, b1_ref, w2t_ref, b2_ref,
          pts_ref, z_ref, outt_ref):
    ft = featst_ref[...]
    sel = sel_ref[...]
    b = lax.dot_general(ft, sel, (((0,), (0,)), ((), ())),
                        preferred_element_type=jnp.float32)
    near = b[:, 6 * _MAX:7 * _MAX]
    fmn = b[:, 7 * _MAX:8 * _MAX]
    dist = b[:, 8 * _MAX:9 * _MAX]

    R = ft.shape[1]
    jf = lax.broadcasted_iota(jnp.int32, (R, _MAX), 1).astype(jnp.float32)

    m25 = dist < 25.0
    m50 = dist < 50.0
    m100 = dist < 100.0
    inv = jnp.where(m25, 1.0 / 127.0,
          jnp.where(m50, 1.0 / 63.0,
          jnp.where(m100, 1.0 / 31.0, 1.0 / 15.0)))
    nsf = jnp.where(m25, 128.0,
          jnp.where(m50, 64.0,
          jnp.where(m100, 32.0, 16.0)))
    live = jf < nsf

    t = jf * inv
    z = near + fmn * t
    z_ref[...] = jnp.where(live, z, 0.0)

    for c in range(3):
        o_c = b[:, c * _MAX:(c + 1) * _MAX]
        d_c = b[:, (c + 3) * _MAX:(c + 4) * _MAX]
        pts_ref[c, :, :] = jnp.where(live, o_c + d_c * z, 0.0)

    h = jnp.maximum(
        jnp.dot(w1t_ref[...], ft, preferred_element_type=jnp.float32) + b1_ref[...],
        0.0)
    outt_ref[...] = jnp.dot(w2t_ref[...], h, preferred_element_type=jnp.float32) + b2_ref[...]


def kernel(rays_o, rays_d, bounds, distances, W1, b1, W2, b2):
    N = rays_o.shape[0]
    feats_t = jnp.concatenate([rays_o.T, rays_d.T, bounds.T, distances[None, :]], axis=0)
    R = _N_BLOCK
    grid = (N // R,)

    pts_t, z_vals, out_t = pl.pallas_call(
        _body,
        grid=grid,
        in_specs=[
            pl.BlockSpec((9, R), lambda i: (0, i)),
            pl.BlockSpec((9, 9 * _MAX), lambda i: (0, 0)),
            pl.BlockSpec((256, 9), lambda i: (0, 0)),
            pl.BlockSpec((256, 1), lambda i: (0, 0)),
            pl.BlockSpec((4, 256), lambda i: (0, 0)),
            pl.BlockSpec((4, 1), lambda i: (0, 0)),
        ],
        out_specs=[
            pl.BlockSpec((3, R, _MAX), lambda i: (0, i, 0)),
            pl.BlockSpec((R, _MAX), lambda i: (i, 0)),
            pl.BlockSpec((4, R), lambda i: (0, i)),
        ],
        out_shape=[
            jax.ShapeDtypeStruct((3, N, _MAX), jnp.float32),
            jax.ShapeDtypeStruct((N, _MAX), jnp.float32),
            jax.ShapeDtypeStruct((4, N), jnp.float32),
        ],
    )(feats_t, jnp.asarray(_SEL), W1.T, b1.reshape(256, 1), W2.T, b2.reshape(4, 1))

    return jnp.transpose(pts_t, (1, 2, 0)), z_vals, out_t.T
